# baseline (speedup 1.0000x reference)
import functools

import jax
import jax.numpy as jnp
from jax import lax
from jax.experimental import pallas as pl
from jax.experimental.pallas import tpu as pltpu
from jax.experimental.pallas import tpu_sc as plsc

O = 10000
M = 1000
D = 128
DP = 144
NODES = 11008
DUMMY = 11000
NE = 320000
NSC = 2
NTILE = 16
K = 80
CHUNKS = 250
GCH = 25
GROUPS = CHUNKS // GCH
TCH = NSC * NTILE * CHUNKS
RPT = NODES // NTILE
BLK = 688
GRID = NODES // BLK


def _sc_segsum(tab, gidx, sidx):
    mesh = plsc.VectorSubcoreMesh(core_axis_name="c", subcore_axis_name="s")

    @functools.partial(
        pl.kernel,
        out_type=jax.ShapeDtypeStruct((NSC * NODES, DP), jnp.float32),
        mesh=mesh,
        scratch_types=[
            pltpu.VMEM_SHARED((NODES, DP), jnp.float32),
            pltpu.VMEM((GCH, K), jnp.int32),
            pltpu.VMEM((GCH, K), jnp.int32),
            pltpu.VMEM((GCH, K), jnp.int32),
            pltpu.VMEM((GCH, K), jnp.int32),
            pltpu.VMEM((K, DP), jnp.float32),
            pltpu.VMEM((K, DP), jnp.float32),
            pltpu.SemaphoreType.DMA,
            pltpu.SemaphoreType.DMA,
            pltpu.SemaphoreType.DMA,
        ],
        compiler_params=pltpu.CompilerParams(use_tc_tiling_on_sc=False),
    )
    def run(tab_hbm, gidx_hbm, sidx_hbm, out_hbm, acc,
            giA, siA, giB, siB, rowsA, rowsB, semA, semB, semI):
        cid = lax.axis_index("c")
        sid = lax.axis_index("s")
        zero16 = jnp.zeros((16,), jnp.float32)

        def zrow(i, carry):
            for j in range(DP // 16):
                rowsA[i, pl.ds(j * 16, 16)] = zero16
            return carry

        lax.fori_loop(0, K, zrow, 0)
        r0 = sid * RPT
        for k in range(RPT // K):
            pltpu.sync_copy(rowsA, acc.at[pl.ds(r0 + k * K, K)])
        rem = RPT % K
        if rem:
            pltpu.sync_copy(rowsA.at[pl.ds(0, rem)],
                            acc.at[pl.ds(r0 + (RPT // K) * K, rem)])
        plsc.subcore_barrier()

        chunk0 = (cid * NTILE + sid) * CHUNKS

        def idx_start(g, gbuf, sbuf):
            row = chunk0 + g * GCH
            pltpu.async_copy(gidx_hbm.at[pl.ds(row, GCH)], gbuf, semI)
            pltpu.async_copy(sidx_hbm.at[pl.ds(row, GCH)], sbuf, semI)

        def idx_wait(g, gbuf, sbuf):
            row = chunk0 + g * GCH
            pltpu.make_async_copy(gidx_hbm.at[pl.ds(row, GCH)], gbuf, semI).wait()
            pltpu.make_async_copy(sidx_hbm.at[pl.ds(row, GCH)], sbuf, semI).wait()

        def gwait(gi, rows, sem):
            pltpu.make_async_copy(tab_hbm.at[gi], rows, sem).wait()

        idx_start(0, giA, siA)
        for g in range(GROUPS):
            gbuf, sbuf = (giA, siA) if g % 2 == 0 else (giB, siB)
            nbuf, mbuf = (giB, siB) if g % 2 == 0 else (giA, siA)
            idx_wait(g, gbuf, sbuf)
            if g + 1 < GROUPS:
                idx_start(g + 1, nbuf, mbuf)
            pltpu.async_copy(tab_hbm.at[gbuf.at[0]], rowsA, semA)

            def body(j, carry):
                pltpu.async_copy(tab_hbm.at[gbuf.at[2 * j + 1]], rowsB, semB)
                gwait(gbuf.at[2 * j], rowsA, semA)
                pltpu.sync_copy(rowsA, acc.at[sbuf.at[2 * j]], add=True)
                pltpu.async_copy(tab_hbm.at[gbuf.at[2 * j + 2]], rowsA, semA)
                gwait(gbuf.at[2 * j + 1], rowsB, semB)
                pltpu.sync_copy(rowsB, acc.at[sbuf.at[2 * j + 1]], add=True)
                return carry

            lax.fori_loop(0, GCH // 2, body, 0)
            gwait(gbuf.at[GCH - 1], rowsA, semA)
            pltpu.sync_copy(rowsA, acc.at[sbuf.at[GCH - 1]], add=True)
        plsc.subcore_barrier()
        pltpu.sync_copy(acc.at[pl.ds(r0, RPT)],
                        out_hbm.at[pl.ds(cid * NODES + r0, RPT)])

    return run(tab, gidx, sidx).reshape(NSC, NODES, DP)


def _ones_col(rows):
    col = lax.broadcasted_iota(jnp.int32, (rows, DP - D), 1)
    return jnp.where(col == 0, jnp.float32(1.0), jnp.float32(0.0))


def _tc_prologue(feat, wu, b_op, b_m):
    def body(feat_ref, wu_ref, bop_ref, bm_ref, out_ref):
        b = pl.program_id(0)
        y = jnp.dot(feat_ref[...], wu_ref[...],
                    preferred_element_type=jnp.float32)
        rid = b * BLK + lax.broadcasted_iota(jnp.int32, (BLK, 1), 0)
        y = y + jnp.where(rid < O, bop_ref[...], bm_ref[...])
        out_ref[:, :D] = y
        out_ref[:, D:] = _ones_col(BLK)

    return pl.pallas_call(
        body,
        grid=(GRID,),
        in_specs=[
            pl.BlockSpec((BLK, 128), lambda b: (b, 0)),
            pl.BlockSpec((128, D), lambda b: (0, 0)),
            pl.BlockSpec((1, D), lambda b: (0, 0)),
            pl.BlockSpec((1, D), lambda b: (0, 0)),
        ],
        out_specs=pl.BlockSpec((BLK, DP), lambda b: (b, 0)),
        out_shape=jax.ShapeDtypeStruct((NODES, DP), jnp.float32),
    )(feat, wu, b_op.reshape(1, D), b_m.reshape(1, D))


def _tc_layer(tab, acc0, acc1, wp, wc, bp, bc, g_op, beta_op, g_m, beta_m):
    def body(tab_ref, a0_ref, a1_ref, wp_ref, wc_ref, bp_ref, bc_ref,
             gop_ref, betop_ref, gm_ref, betm_ref, out_ref):
        b = pl.program_id(0)
        x = tab_ref[:, :D]
        s0 = a0_ref[:, :D]
        c0 = a0_ref[:, D:D + 1]
        s1 = a1_ref[:, :D]
        c1 = a1_ref[:, D:D + 1]
        zero = jnp.float32(0.0)
        y = (x
             + jnp.dot(s0 / jnp.maximum(c0, 1.0), wp_ref[...],
                       preferred_element_type=jnp.float32)
             + jnp.dot(s1 / jnp.maximum(c1, 1.0), wc_ref[...],
                       preferred_element_type=jnp.float32)
             + jnp.where(c0 > 0, bp_ref[...], zero)
             + jnp.where(c1 > 0, bc_ref[...], zero))
        mean = jnp.mean(y, axis=-1, keepdims=True)
        var = jnp.mean(jnp.square(y - mean), axis=-1, keepdims=True)
        rid = b * BLK + lax.broadcasted_iota(jnp.int32, (BLK, 1), 0)
        is_op = rid < O
        g = jnp.where(is_op, gop_ref[...], gm_ref[...])
        beta = jnp.where(is_op, betop_ref[...], betm_ref[...])
        z = (y - mean) * lax.rsqrt(var + 1e-5) * g + beta
        out_ref[:, :D] = z
        out_ref[:, D:] = _ones_col(BLK)

    row = lambda v: v.reshape(1, D)
    full = lambda shape: pl.BlockSpec(shape, lambda b: (0, 0))
    blk = pl.BlockSpec((BLK, DP), lambda b: (b, 0))
    return pl.pallas_call(
        body,
        grid=(GRID,),
        in_specs=[blk, blk, blk, full((D, D)), full((D, D)),
                  full((1, D)), full((1, D)), full((1, D)), full((1, D)),
                  full((1, D)), full((1, D))],
        out_specs=blk,
        out_shape=jax.ShapeDtypeStruct((NODES, DP), jnp.float32),
    )(tab, acc0, acc1, wp, wc, row(bp), row(bc),
      row(g_op), row(beta_op), row(g_m), row(beta_m))


def _prep_edges(prec, compat):
    ps = prec[0].astype(jnp.int32)
    pt = prec[1].astype(jnp.int32)
    cs = compat[0].astype(jnp.int32)
    ct = compat[1].astype(jnp.int32)
    cross = (cs < O) != (ct < O)
    csc = jnp.where(cross, ct, DUMMY)
    pt_s, ps_s = lax.sort((pt, ps), num_keys=1)
    csc_s, cs_s = lax.sort((csc, cs), num_keys=1)
    gidx = jnp.concatenate([ps_s, cs_s]).reshape(TCH, K)
    sidx = jnp.concatenate([pt_s, csc_s]).reshape(TCH, K)
    return gidx, sidx


def kernel(op_features, machine_features, precedence_edges,
           compatibility_edges, W_op, b_op, W_m, b_m, W_prec, b_prec,
           W_comp, b_comp, g_op_ln, beta_op_ln, g_m_ln, beta_m_ln):
    f32 = jnp.float32
    feat = jnp.zeros((NODES, 128), f32)
    feat = feat.at[:O, :6].set(op_features.astype(f32))
    feat = feat.at[O:O + M, 6:8].set(machine_features.astype(f32))
    wu = jnp.zeros((128, D), f32)
    wu = wu.at[:6].set(W_op.astype(f32)).at[6:8].set(W_m.astype(f32))

    gidx, sidx = _prep_edges(precedence_edges, compatibility_edges)
    tab = _tc_prologue(feat, wu, b_op, b_m)
    for l in range(W_prec.shape[0]):
        acc = _sc_segsum(tab, gidx, sidx)
        tab = _tc_layer(tab, acc[0], acc[1], W_prec[l], W_comp[l],
                        b_prec[l], b_comp[l], g_op_ln[l], beta_op_ln[l],
                        g_m_ln[l], beta_m_ln[l])
    return tab[:O, :D], tab[O:O + M, :D]


# speedup vs baseline: 1.6756x; 1.6756x over previous
import functools

import jax
import jax.numpy as jnp
from jax import lax
from jax.experimental import pallas as pl
from jax.experimental.pallas import tpu as pltpu
from jax.experimental.pallas import tpu_sc as plsc

O = 10000
M = 1000
D = 128
DP = 136
NODES = 11008
DUMMY = 11000
NE = 320000
NSC = 2
NTILE = 16
K = 80
CHUNKS = 250
GCH = 25
GROUPS = CHUNKS // GCH
TCH = NSC * NTILE * CHUNKS
RPT = NODES // NTILE
BLK = 688
GRID = NODES // BLK


def _sc_segsum(tab, gidx, sidx):
    mesh = plsc.VectorSubcoreMesh(core_axis_name="c", subcore_axis_name="s")

    @functools.partial(
        pl.kernel,
        out_type=jax.ShapeDtypeStruct((NSC * NODES, DP), jnp.float32),
        mesh=mesh,
        scratch_types=[
            pltpu.VMEM_SHARED((NODES, DP), jnp.float32),
            pltpu.VMEM((GCH, K), jnp.int32),
            pltpu.VMEM((GCH, K), jnp.int32),
            pltpu.VMEM((GCH, K), jnp.int32),
            pltpu.VMEM((GCH, K), jnp.int32),
            pltpu.VMEM((K, DP), jnp.float32),
            pltpu.VMEM((K, DP), jnp.float32),
            pltpu.SemaphoreType.DMA,
            pltpu.SemaphoreType.DMA,
            pltpu.SemaphoreType.DMA,
        ],
        compiler_params=pltpu.CompilerParams(use_tc_tiling_on_sc=False),
    )
    def run(tab_hbm, gidx_hbm, sidx_hbm, out_hbm, acc,
            giA, siA, giB, siB, rowsA, rowsB, semA, semB, semI):
        cid = lax.axis_index("c")
        sid = lax.axis_index("s")
        zero16 = jnp.zeros((16,), jnp.float32)

        def zrow(i, carry):
            for j in range(DP // 16):
                rowsA[i, pl.ds(j * 16, 16)] = zero16
            if DP % 16:
                rowsA[i, pl.ds(DP - 16, 16)] = zero16
            return carry

        lax.fori_loop(0, K, zrow, 0)
        r0 = sid * RPT
        for k in range(RPT // K):
            pltpu.sync_copy(rowsA, acc.at[pl.ds(r0 + k * K, K)])
        rem = RPT % K
        if rem:
            pltpu.sync_copy(rowsA.at[pl.ds(0, rem)],
                            acc.at[pl.ds(r0 + (RPT // K) * K, rem)])
        plsc.subcore_barrier()

        chunk0 = (cid * NTILE + sid) * CHUNKS

        def idx_start(g, gbuf, sbuf):
            row = chunk0 + g * GCH
            pltpu.async_copy(gidx_hbm.at[pl.ds(row, GCH)], gbuf, semI)
            pltpu.async_copy(sidx_hbm.at[pl.ds(row, GCH)], sbuf, semI)

        def idx_wait(g, gbuf, sbuf):
            row = chunk0 + g * GCH
            pltpu.make_async_copy(gidx_hbm.at[pl.ds(row, GCH)], gbuf, semI).wait()
            pltpu.make_async_copy(sidx_hbm.at[pl.ds(row, GCH)], sbuf, semI).wait()

        def gwait(gi, rows, sem):
            pltpu.make_async_copy(tab_hbm.at[gi], rows, sem).wait()

        idx_start(0, giA, siA)
        for g in range(GROUPS):
            gbuf, sbuf = (giA, siA) if g % 2 == 0 else (giB, siB)
            nbuf, mbuf = (giB, siB) if g % 2 == 0 else (giA, siA)
            idx_wait(g, gbuf, sbuf)
            if g + 1 < GROUPS:
                idx_start(g + 1, nbuf, mbuf)
            pltpu.async_copy(tab_hbm.at[gbuf.at[0]], rowsA, semA)

            def body(j, carry):
                pltpu.async_copy(tab_hbm.at[gbuf.at[2 * j + 1]], rowsB, semB)
                gwait(gbuf.at[2 * j], rowsA, semA)
                pltpu.sync_copy(rowsA, acc.at[sbuf.at[2 * j]], add=True)
                pltpu.async_copy(tab_hbm.at[gbuf.at[2 * j + 2]], rowsA, semA)
                gwait(gbuf.at[2 * j + 1], rowsB, semB)
                pltpu.sync_copy(rowsB, acc.at[sbuf.at[2 * j + 1]], add=True)
                return carry

            lax.fori_loop(0, GCH // 2, body, 0)
            gwait(gbuf.at[GCH - 1], rowsA, semA)
            pltpu.sync_copy(rowsA, acc.at[sbuf.at[GCH - 1]], add=True)
        plsc.subcore_barrier()
        pltpu.sync_copy(acc.at[pl.ds(r0, RPT)],
                        out_hbm.at[pl.ds(cid * NODES + r0, RPT)])

    return run(tab, gidx, sidx).reshape(NSC, NODES, DP)


def _ones_col(rows):
    col = lax.broadcasted_iota(jnp.int32, (rows, DP - D), 1)
    return jnp.where(col == 0, jnp.float32(1.0), jnp.float32(0.0))


def _tc_prologue(feat, wu, b_op, b_m):
    def body(feat_ref, wu_ref, bop_ref, bm_ref, out_ref):
        b = pl.program_id(0)
        y = jnp.dot(feat_ref[...], wu_ref[...],
                    preferred_element_type=jnp.float32)
        rid = b * BLK + lax.broadcasted_iota(jnp.int32, (BLK, 1), 0)
        y = y + jnp.where(rid < O, bop_ref[...], bm_ref[...])
        out_ref[:, :D] = y
        out_ref[:, D:] = _ones_col(BLK)

    return pl.pallas_call(
        body,
        grid=(GRID,),
        in_specs=[
            pl.BlockSpec((BLK, 128), lambda b: (b, 0)),
            pl.BlockSpec((128, D), lambda b: (0, 0)),
            pl.BlockSpec((1, D), lambda b: (0, 0)),
            pl.BlockSpec((1, D), lambda b: (0, 0)),
        ],
        out_specs=pl.BlockSpec((BLK, DP), lambda b: (b, 0)),
        out_shape=jax.ShapeDtypeStruct((NODES, DP), jnp.float32),
    )(feat, wu, b_op.reshape(1, D), b_m.reshape(1, D))


def _tc_layer(tab, acc0, acc1, wp, wc, bp, bc, g_op, beta_op, g_m, beta_m):
    def body(tab_ref, a0_ref, a1_ref, wp_ref, wc_ref, bp_ref, bc_ref,
             gop_ref, betop_ref, gm_ref, betm_ref, out_ref):
        b = pl.program_id(0)
        x = tab_ref[:, :D]
        s0 = a0_ref[:, :D]
        c0 = a0_ref[:, D:D + 1]
        s1 = a1_ref[:, :D]
        c1 = a1_ref[:, D:D + 1]
        zero = jnp.float32(0.0)
        y = (x
             + jnp.dot(s0 / jnp.maximum(c0, 1.0), wp_ref[...],
                       preferred_element_type=jnp.float32)
             + jnp.dot(s1 / jnp.maximum(c1, 1.0), wc_ref[...],
                       preferred_element_type=jnp.float32)
             + jnp.where(c0 > 0, bp_ref[...], zero)
             + jnp.where(c1 > 0, bc_ref[...], zero))
        mean = jnp.mean(y, axis=-1, keepdims=True)
        var = jnp.mean(jnp.square(y - mean), axis=-1, keepdims=True)
        rid = b * BLK + lax.broadcasted_iota(jnp.int32, (BLK, 1), 0)
        is_op = rid < O
        g = jnp.where(is_op, gop_ref[...], gm_ref[...])
        beta = jnp.where(is_op, betop_ref[...], betm_ref[...])
        z = (y - mean) * lax.rsqrt(var + 1e-5) * g + beta
        out_ref[:, :D] = z
        out_ref[:, D:] = _ones_col(BLK)

    row = lambda v: v.reshape(1, D)
    full = lambda shape: pl.BlockSpec(shape, lambda b: (0, 0))
    blk = pl.BlockSpec((BLK, DP), lambda b: (b, 0))
    return pl.pallas_call(
        body,
        grid=(GRID,),
        in_specs=[blk, blk, blk, full((D, D)), full((D, D)),
                  full((1, D)), full((1, D)), full((1, D)), full((1, D)),
                  full((1, D)), full((1, D))],
        out_specs=blk,
        out_shape=jax.ShapeDtypeStruct((NODES, DP), jnp.float32),
    )(tab, acc0, acc1, wp, wc, row(bp), row(bc),
      row(g_op), row(beta_op), row(g_m), row(beta_m))


def _prep_edges(prec, compat):
    ps = prec[0].astype(jnp.int32)
    pt = prec[1].astype(jnp.int32)
    cs = compat[0].astype(jnp.int32)
    ct = compat[1].astype(jnp.int32)
    cross = (cs < O) != (ct < O)
    csc = jnp.where(cross, ct, DUMMY)
    gidx = jnp.concatenate([ps, cs]).reshape(TCH, K)
    sidx = jnp.concatenate([pt, csc]).reshape(TCH, K)
    return gidx, sidx


def kernel(op_features, machine_features, precedence_edges,
           compatibility_edges, W_op, b_op, W_m, b_m, W_prec, b_prec,
           W_comp, b_comp, g_op_ln, beta_op_ln, g_m_ln, beta_m_ln):
    f32 = jnp.float32
    feat = jnp.zeros((NODES, 128), f32)
    feat = feat.at[:O, :6].set(op_features.astype(f32))
    feat = feat.at[O:O + M, 6:8].set(machine_features.astype(f32))
    wu = jnp.zeros((128, D), f32)
    wu = wu.at[:6].set(W_op.astype(f32)).at[6:8].set(W_m.astype(f32))

    gidx, sidx = _prep_edges(precedence_edges, compatibility_edges)
    tab = _tc_prologue(feat, wu, b_op, b_m)
    for l in range(W_prec.shape[0]):
        acc = _sc_segsum(tab, gidx, sidx)
        tab = _tc_layer(tab, acc[0], acc[1], W_prec[l], W_comp[l],
                        b_prec[l], b_comp[l], g_op_ln[l], beta_op_ln[l],
                        g_m_ln[l], beta_m_ln[l])
    return tab[:O, :D], tab[O:O + M, :D]


# speedup vs baseline: 1.7075x; 1.0190x over previous
import functools

import jax
import jax.numpy as jnp
from jax import lax
from jax.experimental import pallas as pl
from jax.experimental.pallas import tpu as pltpu
from jax.experimental.pallas import tpu_sc as plsc

O = 10000
M = 1000
D = 128
DP = 144
NODES = 11008
DUMMY = 11000
NE = 320000
NSC = 2
NTILE = 16
K = 125
CHUNKS = 160
GCH = 16
GROUPS = CHUNKS // GCH
TCH = NSC * NTILE * CHUNKS
RPT = NODES // NTILE
BLK = 688
GRID = NODES // BLK


def _sc_segsum(tab, gidx, sidx):
    mesh = plsc.VectorSubcoreMesh(core_axis_name="c", subcore_axis_name="s")

    @functools.partial(
        pl.kernel,
        out_type=jax.ShapeDtypeStruct((NSC * NODES, DP), jnp.float32),
        mesh=mesh,
        scratch_types=[
            pltpu.VMEM_SHARED((NODES, DP), jnp.float32),
            pltpu.VMEM((GCH, K), jnp.int32),
            pltpu.VMEM((GCH, K), jnp.int32),
            pltpu.VMEM((GCH, K), jnp.int32),
            pltpu.VMEM((GCH, K), jnp.int32),
            pltpu.VMEM((K, DP), jnp.float32),
            pltpu.SemaphoreType.DMA,
            pltpu.SemaphoreType.DMA,
        ],
        compiler_params=pltpu.CompilerParams(use_tc_tiling_on_sc=False),
    )
    def run(tab_hbm, gidx_hbm, sidx_hbm, out_hbm, acc,
            giA, siA, giB, siB, rowsA, semA, semI):
        cid = lax.axis_index("c")
        sid = lax.axis_index("s")
        zero16 = jnp.zeros((16,), jnp.float32)

        def zrow(i, carry):
            for j in range(DP // 16):
                rowsA[i, pl.ds(j * 16, 16)] = zero16
            if DP % 16:
                rowsA[i, pl.ds(DP - 16, 16)] = zero16
            return carry

        lax.fori_loop(0, K, zrow, 0)
        r0 = sid * RPT
        for k in range(RPT // K):
            pltpu.sync_copy(rowsA, acc.at[pl.ds(r0 + k * K, K)])
        rem = RPT % K
        if rem:
            pltpu.sync_copy(rowsA.at[pl.ds(0, rem)],
                            acc.at[pl.ds(r0 + (RPT // K) * K, rem)])
        plsc.subcore_barrier()

        chunk0 = (cid * NTILE + sid) * CHUNKS

        def idx_start(g, gbuf, sbuf):
            row = chunk0 + g * GCH
            pltpu.async_copy(gidx_hbm.at[pl.ds(row, GCH)], gbuf, semI)
            pltpu.async_copy(sidx_hbm.at[pl.ds(row, GCH)], sbuf, semI)

        def idx_wait(g, gbuf, sbuf):
            row = chunk0 + g * GCH
            pltpu.make_async_copy(gidx_hbm.at[pl.ds(row, GCH)], gbuf, semI).wait()
            pltpu.make_async_copy(sidx_hbm.at[pl.ds(row, GCH)], sbuf, semI).wait()

        def gwait(gi, rows, sem):
            pltpu.make_async_copy(tab_hbm.at[gi], rows, sem).wait()

        idx_start(0, giA, siA)
        for g in range(GROUPS):
            gbuf, sbuf = (giA, siA) if g % 2 == 0 else (giB, siB)
            nbuf, mbuf = (giB, siB) if g % 2 == 0 else (giA, siA)
            idx_wait(g, gbuf, sbuf)
            if g + 1 < GROUPS:
                idx_start(g + 1, nbuf, mbuf)

            def body(j, carry):
                pltpu.async_copy(tab_hbm.at[gbuf.at[j]], rowsA, semA)
                gwait(gbuf.at[j], rowsA, semA)
                pltpu.sync_copy(rowsA, acc.at[sbuf.at[j]], add=True)
                return carry

            lax.fori_loop(0, GCH, body, 0)
        plsc.subcore_barrier()
        pltpu.sync_copy(acc.at[pl.ds(r0, RPT)],
                        out_hbm.at[pl.ds(cid * NODES + r0, RPT)])

    return run(tab, gidx, sidx).reshape(NSC, NODES, DP)


def _ones_col(rows):
    col = lax.broadcasted_iota(jnp.int32, (rows, DP - D), 1)
    return jnp.where(col == 0, jnp.float32(1.0), jnp.float32(0.0))


def _tc_prologue(feat, wu, b_op, b_m):
    def body(feat_ref, wu_ref, bop_ref, bm_ref, out_ref):
        b = pl.program_id(0)
        y = jnp.dot(feat_ref[...], wu_ref[...],
                    preferred_element_type=jnp.float32)
        rid = b * BLK + lax.broadcasted_iota(jnp.int32, (BLK, 1), 0)
        y = y + jnp.where(rid < O, bop_ref[...], bm_ref[...])
        out_ref[:, :D] = y
        out_ref[:, D:] = _ones_col(BLK)

    return pl.pallas_call(
        body,
        grid=(GRID,),
        in_specs=[
            pl.BlockSpec((BLK, 128), lambda b: (b, 0)),
            pl.BlockSpec((128, D), lambda b: (0, 0)),
            pl.BlockSpec((1, D), lambda b: (0, 0)),
            pl.BlockSpec((1, D), lambda b: (0, 0)),
        ],
        out_specs=pl.BlockSpec((BLK, DP), lambda b: (b, 0)),
        out_shape=jax.ShapeDtypeStruct((NODES, DP), jnp.float32),
    )(feat, wu, b_op.reshape(1, D), b_m.reshape(1, D))


def _tc_layer(tab, acc0, acc1, wp, wc, bp, bc, g_op, beta_op, g_m, beta_m):
    def body(tab_ref, a0_ref, a1_ref, wp_ref, wc_ref, bp_ref, bc_ref,
             gop_ref, betop_ref, gm_ref, betm_ref, out_ref):
        b = pl.program_id(0)
        x = tab_ref[:, :D]
        s0 = a0_ref[:, :D]
        c0 = a0_ref[:, D:D + 1]
        s1 = a1_ref[:, :D]
        c1 = a1_ref[:, D:D + 1]
        zero = jnp.float32(0.0)
        y = (x
             + jnp.dot(s0 / jnp.maximum(c0, 1.0), wp_ref[...],
                       preferred_element_type=jnp.float32)
             + jnp.dot(s1 / jnp.maximum(c1, 1.0), wc_ref[...],
                       preferred_element_type=jnp.float32)
             + jnp.where(c0 > 0, bp_ref[...], zero)
             + jnp.where(c1 > 0, bc_ref[...], zero))
        mean = jnp.mean(y, axis=-1, keepdims=True)
        var = jnp.mean(jnp.square(y - mean), axis=-1, keepdims=True)
        rid = b * BLK + lax.broadcasted_iota(jnp.int32, (BLK, 1), 0)
        is_op = rid < O
        g = jnp.where(is_op, gop_ref[...], gm_ref[...])
        beta = jnp.where(is_op, betop_ref[...], betm_ref[...])
        z = (y - mean) * lax.rsqrt(var + 1e-5) * g + beta
        out_ref[:, :D] = z
        out_ref[:, D:] = _ones_col(BLK)

    row = lambda v: v.reshape(1, D)
    full = lambda shape: pl.BlockSpec(shape, lambda b: (0, 0))
    blk = pl.BlockSpec((BLK, DP), lambda b: (b, 0))
    return pl.pallas_call(
        body,
        grid=(GRID,),
        in_specs=[blk, blk, blk, full((D, D)), full((D, D)),
                  full((1, D)), full((1, D)), full((1, D)), full((1, D)),
                  full((1, D)), full((1, D))],
        out_specs=blk,
        out_shape=jax.ShapeDtypeStruct((NODES, DP), jnp.float32),
    )(tab, acc0, acc1, wp, wc, row(bp), row(bc),
      row(g_op), row(beta_op), row(g_m), row(beta_m))


def _prep_edges(prec, compat):
    ps = prec[0].astype(jnp.int32)
    pt = prec[1].astype(jnp.int32)
    cs = compat[0].astype(jnp.int32)
    ct = compat[1].astype(jnp.int32)
    cross = (cs < O) != (ct < O)
    csc = jnp.where(cross, ct, DUMMY)
    gidx = jnp.concatenate([ps, cs]).reshape(TCH, K)
    sidx = jnp.concatenate([pt, csc]).reshape(TCH, K)
    return gidx, sidx


def kernel(op_features, machine_features, precedence_edges,
           compatibility_edges, W_op, b_op, W_m, b_m, W_prec, b_prec,
           W_comp, b_comp, g_op_ln, beta_op_ln, g_m_ln, beta_m_ln):
    f32 = jnp.float32
    feat = jnp.zeros((NODES, 128), f32)
    feat = feat.at[:O, :6].set(op_features.astype(f32))
    feat = feat.at[O:O + M, 6:8].set(machine_features.astype(f32))
    wu = jnp.zeros((128, D), f32)
    wu = wu.at[:6].set(W_op.astype(f32)).at[6:8].set(W_m.astype(f32))

    gidx, sidx = _prep_edges(precedence_edges, compatibility_edges)
    tab = _tc_prologue(feat, wu, b_op, b_m)
    for l in range(W_prec.shape[0]):
        acc = _sc_segsum(tab, gidx, sidx)
        tab = _tc_layer(tab, acc[0], acc[1], W_prec[l], W_comp[l],
                        b_prec[l], b_comp[l], g_op_ln[l], beta_op_ln[l],
                        g_m_ln[l], beta_m_ln[l])
    return tab[:O, :D], tab[O:O + M, :D]


# speedup vs baseline: 1.7289x; 1.0125x over previous
import functools

import jax
import jax.numpy as jnp
from jax import lax
from jax.experimental import pallas as pl
from jax.experimental.pallas import tpu as pltpu
from jax.experimental.pallas import tpu_sc as plsc

O = 10000
M = 1000
D = 128
DP = 144
NODES = 11008
DUMMY = 11000
NE = 320000
NSC = 2
NTILE = 16
K = 80
CHUNKS = 250
GCH = 25
GROUPS = CHUNKS // GCH
TCH = NSC * NTILE * CHUNKS
RPT = NODES // NTILE
BLK = 688
GRID = NODES // BLK


def _sc_segsum(tab, gidx, sidx):
    mesh = plsc.VectorSubcoreMesh(core_axis_name="c", subcore_axis_name="s")

    @functools.partial(
        pl.kernel,
        out_type=jax.ShapeDtypeStruct((NSC * NODES, DP), jnp.float32),
        mesh=mesh,
        scratch_types=[
            pltpu.VMEM_SHARED((NODES, DP), jnp.float32),
            pltpu.VMEM((GCH, K), jnp.int32),
            pltpu.VMEM((GCH, K), jnp.int32),
            pltpu.VMEM((GCH, K), jnp.int32),
            pltpu.VMEM((GCH, K), jnp.int32),
            pltpu.VMEM((K, DP), jnp.float32),
            pltpu.VMEM((K, DP), jnp.float32),
            pltpu.SemaphoreType.DMA,
            pltpu.SemaphoreType.DMA,
            pltpu.SemaphoreType.DMA,
        ],
        compiler_params=pltpu.CompilerParams(use_tc_tiling_on_sc=False),
    )
    def run(tab_hbm, gidx_hbm, sidx_hbm, out_hbm, acc,
            giA, siA, giB, siB, rowsA, rowsB, semA, semB, semI):
        cid = lax.axis_index("c")
        sid = lax.axis_index("s")
        zero16 = jnp.zeros((16,), jnp.float32)

        def zrow(i, carry):
            for j in range(DP // 16):
                rowsA[i, pl.ds(j * 16, 16)] = zero16
            if DP % 16:
                rowsA[i, pl.ds(DP - 16, 16)] = zero16
            return carry

        lax.fori_loop(0, K, zrow, 0)
        r0 = sid * RPT
        for k in range(RPT // K):
            pltpu.sync_copy(rowsA, acc.at[pl.ds(r0 + k * K, K)])
        rem = RPT % K
        if rem:
            pltpu.sync_copy(rowsA.at[pl.ds(0, rem)],
                            acc.at[pl.ds(r0 + (RPT // K) * K, rem)])
        plsc.subcore_barrier()

        chunk0 = (cid * NTILE + sid) * CHUNKS

        def idx_start(g, gbuf, sbuf):
            row = chunk0 + g * GCH
            pltpu.async_copy(gidx_hbm.at[pl.ds(row, GCH)], gbuf, semI)
            pltpu.async_copy(sidx_hbm.at[pl.ds(row, GCH)], sbuf, semI)

        def idx_wait(g, gbuf, sbuf):
            row = chunk0 + g * GCH
            pltpu.make_async_copy(gidx_hbm.at[pl.ds(row, GCH)], gbuf, semI).wait()
            pltpu.make_async_copy(sidx_hbm.at[pl.ds(row, GCH)], sbuf, semI).wait()

        def gwait(gi, rows, sem):
            pltpu.make_async_copy(tab_hbm.at[gi], rows, sem).wait()

        idx_start(0, giA, siA)
        for g in range(GROUPS):
            gbuf, sbuf = (giA, siA) if g % 2 == 0 else (giB, siB)
            nbuf, mbuf = (giB, siB) if g % 2 == 0 else (giA, siA)
            idx_wait(g, gbuf, sbuf)
            if g + 1 < GROUPS:
                idx_start(g + 1, nbuf, mbuf)
            pltpu.async_copy(tab_hbm.at[gbuf.at[0]], rowsA, semA)

            def body(j, carry):
                pltpu.async_copy(tab_hbm.at[gbuf.at[2 * j + 1]], rowsB, semB)
                gwait(gbuf.at[2 * j], rowsA, semA)
                pltpu.sync_copy(rowsA, acc.at[sbuf.at[2 * j]], add=True)
                pltpu.async_copy(tab_hbm.at[gbuf.at[2 * j + 2]], rowsA, semA)
                gwait(gbuf.at[2 * j + 1], rowsB, semB)
                pltpu.sync_copy(rowsB, acc.at[sbuf.at[2 * j + 1]], add=True)
                return carry

            lax.fori_loop(0, GCH // 2, body, 0)
            gwait(gbuf.at[GCH - 1], rowsA, semA)
            pltpu.sync_copy(rowsA, acc.at[sbuf.at[GCH - 1]], add=True)
        plsc.subcore_barrier()
        pltpu.sync_copy(acc.at[pl.ds(r0, RPT)],
                        out_hbm.at[pl.ds(cid * NODES + r0, RPT)])

    return run(tab, gidx, sidx).reshape(NSC, NODES, DP)


def _ones_col(rows):
    col = lax.broadcasted_iota(jnp.int32, (rows, DP - D), 1)
    return jnp.where(col == 0, jnp.float32(1.0), jnp.float32(0.0))


def _tc_prologue(feat, wu, b_op, b_m):
    def body(feat_ref, wu_ref, bop_ref, bm_ref, out_ref):
        b = pl.program_id(0)
        y = jnp.dot(feat_ref[...], wu_ref[...],
                    preferred_element_type=jnp.float32)
        rid = b * BLK + lax.broadcasted_iota(jnp.int32, (BLK, 1), 0)
        y = y + jnp.where(rid < O, bop_ref[...], bm_ref[...])
        out_ref[:, :D] = y
        out_ref[:, D:] = _ones_col(BLK)

    return pl.pallas_call(
        body,
        grid=(GRID,),
        in_specs=[
            pl.BlockSpec((BLK, 128), lambda b: (b, 0)),
            pl.BlockSpec((128, D), lambda b: (0, 0)),
            pl.BlockSpec((1, D), lambda b: (0, 0)),
            pl.BlockSpec((1, D), lambda b: (0, 0)),
        ],
        out_specs=pl.BlockSpec((BLK, DP), lambda b: (b, 0)),
        out_shape=jax.ShapeDtypeStruct((NODES, DP), jnp.float32),
    )(feat, wu, b_op.reshape(1, D), b_m.reshape(1, D))


def _tc_layer(tab, acc0, acc1, wp, wc, bp, bc, g_op, beta_op, g_m, beta_m):
    def body(tab_ref, a0_ref, a1_ref, wp_ref, wc_ref, bp_ref, bc_ref,
             gop_ref, betop_ref, gm_ref, betm_ref, out_ref):
        b = pl.program_id(0)
        x = tab_ref[:, :D]
        s0 = a0_ref[:, :D]
        c0 = a0_ref[:, D:D + 1]
        s1 = a1_ref[:, :D]
        c1 = a1_ref[:, D:D + 1]
        zero = jnp.float32(0.0)
        y = (x
             + jnp.dot(s0 / jnp.maximum(c0, 1.0), wp_ref[...],
                       preferred_element_type=jnp.float32)
             + jnp.dot(s1 / jnp.maximum(c1, 1.0), wc_ref[...],
                       preferred_element_type=jnp.float32)
             + jnp.where(c0 > 0, bp_ref[...], zero)
             + jnp.where(c1 > 0, bc_ref[...], zero))
        mean = jnp.mean(y, axis=-1, keepdims=True)
        var = jnp.mean(jnp.square(y - mean), axis=-1, keepdims=True)
        rid = b * BLK + lax.broadcasted_iota(jnp.int32, (BLK, 1), 0)
        is_op = rid < O
        g = jnp.where(is_op, gop_ref[...], gm_ref[...])
        beta = jnp.where(is_op, betop_ref[...], betm_ref[...])
        z = (y - mean) * lax.rsqrt(var + 1e-5) * g + beta
        out_ref[:, :D] = z
        out_ref[:, D:] = _ones_col(BLK)

    row = lambda v: v.reshape(1, D)
    full = lambda shape: pl.BlockSpec(shape, lambda b: (0, 0))
    blk = pl.BlockSpec((BLK, DP), lambda b: (b, 0))
    return pl.pallas_call(
        body,
        grid=(GRID,),
        in_specs=[blk, blk, blk, full((D, D)), full((D, D)),
                  full((1, D)), full((1, D)), full((1, D)), full((1, D)),
                  full((1, D)), full((1, D))],
        out_specs=blk,
        out_shape=jax.ShapeDtypeStruct((NODES, DP), jnp.float32),
    )(tab, acc0, acc1, wp, wc, row(bp), row(bc),
      row(g_op), row(beta_op), row(g_m), row(beta_m))


def _prep_edges(prec, compat):
    ps = prec[0].astype(jnp.int32)
    pt = prec[1].astype(jnp.int32)
    cs = compat[0].astype(jnp.int32)
    ct = compat[1].astype(jnp.int32)
    cross = (cs < O) != (ct < O)
    csc = jnp.where(cross, ct, DUMMY)
    gidx = jnp.concatenate([ps, cs]).reshape(TCH, K)
    sidx = jnp.concatenate([pt, csc]).reshape(TCH, K)
    return gidx, sidx


def kernel(op_features, machine_features, precedence_edges,
           compatibility_edges, W_op, b_op, W_m, b_m, W_prec, b_prec,
           W_comp, b_comp, g_op_ln, beta_op_ln, g_m_ln, beta_m_ln):
    f32 = jnp.float32
    feat = jnp.zeros((NODES, 128), f32)
    feat = feat.at[:O, :6].set(op_features.astype(f32))
    feat = feat.at[O:O + M, 6:8].set(machine_features.astype(f32))
    wu = jnp.zeros((128, D), f32)
    wu = wu.at[:6].set(W_op.astype(f32)).at[6:8].set(W_m.astype(f32))

    gidx, sidx = _prep_edges(precedence_edges, compatibility_edges)
    tab = _tc_prologue(feat, wu, b_op, b_m)
    for l in range(W_prec.shape[0]):
        acc = _sc_segsum(tab, gidx, sidx)
        tab = _tc_layer(tab, acc[0], acc[1], W_prec[l], W_comp[l],
                        b_prec[l], b_comp[l], g_op_ln[l], beta_op_ln[l],
                        g_m_ln[l], beta_m_ln[l])
    return tab[:O, :D], tab[O:O + M, :D]
